# Initial kernel scaffold; baseline (speedup 1.0000x reference)
import jax
import jax.numpy as jnp
from jax import lax
from jax.experimental import pallas as pl
from jax.experimental.pallas import tpu as pltpu
from jax.experimental.pallas import tpu_sc as plsc

NC = 2
NS = 16
NW = NC * NS
L = 16
B_E = 128


def _seg_sum(h, src3, dst3, w3):
  N, D = h.shape
  NB = src3.shape[1]
  assert N % NS == 0 and D % L == 0 and NB % 2 == 0
  RPT = N // NS
  ZR = 125
  assert RPT % ZR == 0

  mesh = plsc.VectorSubcoreMesh(core_axis_name="c", subcore_axis_name="s")

  def body(h_hbm, src_hbm, dst_hbm, w_hbm, out_hbm,
           src_v, dst_v, w_v, rows_a, rows_b, zbuf, acc, sem_a, sem_b):
    cid = lax.axis_index("c")
    sid = lax.axis_index("s")
    wid = sid * NC + cid

    z16 = jnp.zeros((L,), jnp.float32)

    @pl.loop(0, ZR)
    def _(i):
      for c in range(D // L):
        zbuf[i, pl.ds(c * L, L)] = z16

    for k in range(RPT // ZR):
      pltpu.sync_copy(zbuf, acc.at[pl.ds(sid * RPT + k * ZR, ZR)])

    pltpu.sync_copy(src_hbm.at[wid], src_v)
    pltpu.sync_copy(dst_hbm.at[wid], dst_v)
    pltpu.sync_copy(w_hbm.at[wid], w_v)

    plsc.subcore_barrier()

    def scale(rows, j):
      @pl.loop(0, B_E, unroll=4)
      def _(r):
        wv = jnp.full((L,), w_v[j, r])
        for c in range(D // L):
          sl = pl.ds(c * L, L)
          rows[r, sl] = rows[r, sl] * wv

    pltpu.async_copy(h_hbm.at[src_v.at[0]], rows_a, sem_a)
    pltpu.async_copy(h_hbm.at[src_v.at[1]], rows_b, sem_b)

    @pl.loop(0, NB, step=2)
    def _(j):
      for (buf, sem, jj) in ((rows_a, sem_a, j), (rows_b, sem_b, j + 1)):
        pltpu.make_async_copy(h_hbm.at[src_v.at[jj]], buf, sem).wait()
        scale(buf, jj)
        pltpu.sync_copy(buf, acc.at[dst_v.at[jj]], add=True)

        @pl.when(jj + 2 < NB)
        def _():
          pltpu.async_copy(h_hbm.at[src_v.at[jj + 2]], buf, sem)

    plsc.subcore_barrier()
    pltpu.sync_copy(acc.at[pl.ds(sid * RPT, RPT)],
                    out_hbm.at[cid, pl.ds(sid * RPT, RPT)])

  kern = pl.kernel(
      body,
      out_type=jax.ShapeDtypeStruct((NC, N, D), jnp.float32),
      mesh=mesh,
      scratch_types=[
          pltpu.VMEM((NB, B_E), jnp.int32),
          pltpu.VMEM((NB, B_E), jnp.int32),
          pltpu.VMEM((NB, B_E), jnp.float32),
          pltpu.VMEM((B_E, D), jnp.float32),
          pltpu.VMEM((B_E, D), jnp.float32),
          pltpu.VMEM((ZR, D), jnp.float32),
          pltpu.VMEM_SHARED((N, D), jnp.float32),
          pltpu.SemaphoreType.DMA,
          pltpu.SemaphoreType.DMA,
      ],
  )
  return kern(h, src3, dst3, w3)


def _tc_layer(parts, W, b2d, block_n):
  _, N, Din = parts.shape
  Hout = W.shape[1]
  assert N % block_n == 0

  def body(p_ref, w_ref, b_ref, o_ref):
    x = p_ref[0] + p_ref[1]
    y = jnp.dot(x, w_ref[...], preferred_element_type=jnp.float32) + b_ref[...]
    o_ref[...] = jnp.maximum(y, 0.0)

  return pl.pallas_call(
      body,
      grid=(N // block_n,),
      in_specs=[
          pl.BlockSpec((NC, block_n, Din), lambda i: (0, i, 0)),
          pl.BlockSpec((Din, Hout), lambda i: (0, 0)),
          pl.BlockSpec((1, Hout), lambda i: (0, 0)),
      ],
      out_specs=pl.BlockSpec((block_n, Hout), lambda i: (i, 0)),
      out_shape=jax.ShapeDtypeStruct((N, Hout), jnp.float32),
  )(parts, W, b2d)


def _tc_layer2(parts, W1, b2d, W2, block_n):
  _, N, Din = parts.shape
  Hmid = W1.shape[1]
  Cout = W2.shape[1]
  assert N % block_n == 0

  def body(p_ref, w1_ref, b_ref, w2_ref, o_ref):
    x = p_ref[0] + p_ref[1]
    h = jnp.maximum(
        jnp.dot(x, w1_ref[...], preferred_element_type=jnp.float32)
        + b_ref[...], 0.0)
    o_ref[...] = jnp.dot(h, w2_ref[...], preferred_element_type=jnp.float32)

  return pl.pallas_call(
      body,
      grid=(N // block_n,),
      in_specs=[
          pl.BlockSpec((NC, block_n, Din), lambda i: (0, i, 0)),
          pl.BlockSpec((Din, Hmid), lambda i: (0, 0)),
          pl.BlockSpec((1, Hmid), lambda i: (0, 0)),
          pl.BlockSpec((Hmid, Cout), lambda i: (0, 0)),
      ],
      out_specs=pl.BlockSpec((block_n, Cout), lambda i: (i, 0)),
      out_shape=jax.ShapeDtypeStruct((N, Cout), jnp.float32),
  )(parts, W1, b2d, W2)


def _tc_loss(parts, b2d, labels2d):
  _, N, C = parts.shape

  def body(p_ref, b_ref, l_ref, o_ref):
    x = p_ref[0] + p_ref[1] + b_ref[...]
    m = jnp.max(x, axis=1, keepdims=True)
    lse = jnp.log(jnp.sum(jnp.exp(x - m), axis=1, keepdims=True)) + m
    ids = lax.broadcasted_iota(jnp.int32, (N, C), 1)
    picked = jnp.sum(jnp.where(ids == l_ref[...], x, 0.0), axis=1,
                     keepdims=True)
    o_ref[0, 0] = jnp.sum(lse - picked) / N

  out = pl.pallas_call(
      body,
      out_shape=jax.ShapeDtypeStruct((1, 1), jnp.float32),
  )(parts, b2d, labels2d)
  return out[0, 0]


@jax.jit
def kernel(features, edge_index, edge_weight, labels, W0, b0, W1, b1, W2, b2):
  E = edge_weight.shape[0]

  NB = -(-(-(-E // NW)) // B_E)
  NB += NB % 2
  total = NW * NB * B_E
  pad = total - E
  src3 = jnp.pad(edge_index[0], (0, pad)).reshape(NW, NB, B_E)
  dst3 = jnp.pad(edge_index[1], (0, pad)).reshape(NW, NB, B_E)
  w3 = jnp.pad(edge_weight, (0, pad)).reshape(NW, NB, B_E)

  a0 = _seg_sum(features, src3, dst3, w3)
  h1 = _tc_layer(a0, W0, b0.reshape(1, -1), 2000)
  a1 = _seg_sum(h1, src3, dst3, w3)
  p = _tc_layer2(a1, W1, b1.reshape(1, -1), W2, 2000)
  a2 = _seg_sum(p, src3, dst3, w3)
  return _tc_loss(a2, b2.reshape(1, -1),
                  labels.reshape(-1, 1).astype(jnp.int32))


# speedup vs baseline: 3.4650x; 3.4650x over previous
import jax
import jax.numpy as jnp
from jax import lax
from jax.experimental import pallas as pl
from jax.experimental.pallas import tpu as pltpu
from jax.experimental.pallas import tpu_sc as plsc

NC = 2
NS = 16
NW = NC * NS
L = 16
B_E = 128


def _seg_sum(h, src3, dst3, w3, np_rows):
  _, D = h.shape
  NB = src3.shape[1]
  CH = 16
  assert D % L == 0 and NB % CH == 0
  RPT = np_rows // NS
  assert RPT % B_E == 0

  mesh = plsc.VectorSubcoreMesh(core_axis_name="c", subcore_axis_name="s")

  def body(h_hbm, src_hbm, dst_hbm, w_hbm, out_hbm,
           src_v, dst_v, w_v, rows_a, rows_b, acc, sem_a, sem_b):
    cid = lax.axis_index("c")
    sid = lax.axis_index("s")
    wid = sid * NC + cid

    z16 = jnp.zeros((L,), jnp.float32)

    @pl.loop(0, B_E)
    def _(i):
      for c in range(D // L):
        rows_a[i, pl.ds(c * L, L)] = z16

    for k in range(RPT // B_E):
      pltpu.sync_copy(rows_a, acc.at[pl.ds(sid * RPT + k * B_E, B_E)])

    plsc.subcore_barrier()

    def scale(rows, j):
      @pl.loop(0, B_E // L)
      def _(rb):
        wchunk = w_v[j, pl.ds(rb * L, L)]
        for i in range(L):
          wv = jnp.full((L,), wchunk[i])
          r = rb * L + i
          for c in range(D // L):
            sl = pl.ds(c * L, L)
            rows[r, sl] = rows[r, sl] * wv

    @pl.loop(0, NB // CH)
    def _(g):
      pltpu.sync_copy(src_hbm.at[wid, pl.ds(g * CH, CH)], src_v)
      pltpu.sync_copy(dst_hbm.at[wid, pl.ds(g * CH, CH)], dst_v)
      pltpu.sync_copy(w_hbm.at[wid, pl.ds(g * CH, CH)], w_v)

      pltpu.async_copy(h_hbm.at[src_v.at[0]], rows_a, sem_a)
      pltpu.async_copy(h_hbm.at[src_v.at[1]], rows_b, sem_b)

      @pl.loop(0, CH, step=2)
      def _(j):
        for (buf, sem, jj) in ((rows_a, sem_a, j), (rows_b, sem_b, j + 1)):
          pltpu.make_async_copy(h_hbm.at[src_v.at[jj]], buf, sem).wait()
          scale(buf, jj)
          pltpu.sync_copy(buf, acc.at[dst_v.at[jj]], add=True)

          @pl.when(jj + 2 < CH)
          def _():
            pltpu.async_copy(h_hbm.at[src_v.at[jj + 2]], buf, sem)

    plsc.subcore_barrier()
    pltpu.sync_copy(acc.at[pl.ds(sid * RPT, RPT)],
                    out_hbm.at[cid, pl.ds(sid * RPT, RPT)])

  kern = pl.kernel(
      body,
      out_type=jax.ShapeDtypeStruct((NC, np_rows, D), jnp.float32),
      mesh=mesh,
      scratch_types=[
          pltpu.VMEM((CH, B_E), jnp.int32),
          pltpu.VMEM((CH, B_E), jnp.int32),
          pltpu.VMEM((CH, B_E), jnp.float32),
          pltpu.VMEM((B_E, D), jnp.float32),
          pltpu.VMEM((B_E, D), jnp.float32),
          pltpu.VMEM_SHARED((np_rows, D), jnp.float32),
          pltpu.SemaphoreType.DMA,
          pltpu.SemaphoreType.DMA,
      ],
  )
  return kern(h, src3, dst3, w3)


def _tc_layer(parts, W, b2d, block_n):
  _, N, Din = parts.shape
  Hout = W.shape[1]
  assert N % block_n == 0

  def body(p_ref, w_ref, b_ref, o_ref):
    x = p_ref[0] + p_ref[1]
    y = jnp.dot(x, w_ref[...], preferred_element_type=jnp.float32) + b_ref[...]
    o_ref[...] = jnp.maximum(y, 0.0)

  return pl.pallas_call(
      body,
      grid=(N // block_n,),
      in_specs=[
          pl.BlockSpec((NC, block_n, Din), lambda i: (0, i, 0)),
          pl.BlockSpec((Din, Hout), lambda i: (0, 0)),
          pl.BlockSpec((1, Hout), lambda i: (0, 0)),
      ],
      out_specs=pl.BlockSpec((block_n, Hout), lambda i: (i, 0)),
      out_shape=jax.ShapeDtypeStruct((N, Hout), jnp.float32),
  )(parts, W, b2d)


def _tc_loss(parts, W2, b2d, labels2d, n_valid):
  _, NP, _ = parts.shape
  C = W2.shape[1]

  def body(p_ref, w_ref, b_ref, l_ref, o_ref):
    x = jnp.dot(p_ref[0] + p_ref[1], w_ref[...],
                preferred_element_type=jnp.float32) + b_ref[...]
    m = jnp.max(x, axis=1, keepdims=True)
    lse = jnp.log(jnp.sum(jnp.exp(x - m), axis=1, keepdims=True)) + m
    ids = lax.broadcasted_iota(jnp.int32, (NP, C), 1)
    picked = jnp.sum(jnp.where(ids == l_ref[...], x, 0.0), axis=1,
                     keepdims=True)
    rows = lax.broadcasted_iota(jnp.int32, (NP, 1), 0)
    nll = jnp.where(rows < n_valid, lse - picked, 0.0)
    o_ref[...] = jnp.sum(nll, keepdims=True) / n_valid

  out = pl.pallas_call(
      body,
      out_shape=jax.ShapeDtypeStruct((1, 1), jnp.float32),
  )(parts, W2, b2d, labels2d)
  return out[0, 0]


@jax.jit
def kernel(features, edge_index, edge_weight, labels, W0, b0, W1, b1, W2, b2):
  N = features.shape[0]
  E = edge_weight.shape[0]
  NP = NS * 128 * -(--(-N // NS) // 128)

  NB = -(-(-(-E // NW)) // B_E)
  NB += NB % 2
  total = NW * NB * B_E
  pad = total - E
  src3 = jnp.pad(edge_index[0], (0, pad)).reshape(NW, NB, B_E)
  dst3 = jnp.pad(edge_index[1], (0, pad)).reshape(NW, NB, B_E)
  w3 = jnp.pad(edge_weight, (0, pad)).reshape(NW, NB, B_E)

  labels_p = jnp.pad(labels.astype(jnp.int32), (0, NP - N)).reshape(-1, 1)

  a0 = _seg_sum(features, src3, dst3, w3, NP)
  h1 = _tc_layer(a0, W0, b0.reshape(1, -1), 2048)
  a1 = _seg_sum(h1, src3, dst3, w3, NP)
  h2 = _tc_layer(a1, W1, b1.reshape(1, -1), 2048)
  a2 = _seg_sum(h2, src3, dst3, w3, NP)
  return _tc_loss(a2, W2, b2.reshape(1, -1), labels_p, N)


# speedup vs baseline: 3.9164x; 1.1303x over previous
import jax
import jax.numpy as jnp
from jax import lax
from jax.experimental import pallas as pl
from jax.experimental.pallas import tpu as pltpu
from jax.experimental.pallas import tpu_sc as plsc

NC = 2
NS = 16
NW = NC * NS
L = 16
B_E = 128


def _seg_sum(h, src3, dst3, w3, np_rows):
  _, D = h.shape
  NB = src3.shape[1]
  CH = 16
  assert D % L == 0 and NB % CH == 0
  RPT = np_rows // NS
  assert RPT % B_E == 0

  mesh = plsc.VectorSubcoreMesh(core_axis_name="c", subcore_axis_name="s")

  def body(h_hbm, src_hbm, dst_hbm, w_hbm, out_hbm,
           src_v, dst_v, w_v, rows_a, rows_b, acc, sem_a, sem_b,
           sem_sa, sem_sb):
    cid = lax.axis_index("c")
    sid = lax.axis_index("s")
    wid = sid * NC + cid

    z16 = jnp.zeros((L,), jnp.float32)

    @pl.loop(0, B_E)
    def _(i):
      for c in range(D // L):
        rows_a[i, pl.ds(c * L, L)] = z16

    for k in range(RPT // B_E):
      pltpu.sync_copy(rows_a, acc.at[pl.ds(sid * RPT + k * B_E, B_E)])

    plsc.subcore_barrier()

    def scale(rows, j):
      @pl.loop(0, B_E // L)
      def _(rb):
        wchunk = w_v[j, pl.ds(rb * L, L)]
        for i in range(L):
          wv = jnp.full((L,), wchunk[i])
          r = rb * L + i
          for c in range(D // L):
            sl = pl.ds(c * L, L)
            rows[r, sl] = rows[r, sl] * wv

    @pl.loop(0, NB // CH)
    def _(g):
      pltpu.sync_copy(src_hbm.at[wid, pl.ds(g * CH, CH)], src_v)
      pltpu.sync_copy(dst_hbm.at[wid, pl.ds(g * CH, CH)], dst_v)
      pltpu.sync_copy(w_hbm.at[wid, pl.ds(g * CH, CH)], w_v)

      pltpu.async_copy(h_hbm.at[src_v.at[0]], rows_a, sem_a)
      pltpu.async_copy(h_hbm.at[src_v.at[1]], rows_b, sem_b)

      @pl.loop(0, CH, step=2)
      def _(j):
        for (buf, sem, ssem, jj) in ((rows_a, sem_a, sem_sa, j),
                                     (rows_b, sem_b, sem_sb, j + 1)):
          pltpu.make_async_copy(h_hbm.at[src_v.at[jj]], buf, sem).wait()
          scale(buf, jj)
          pltpu.async_copy(buf, acc.at[dst_v.at[jj]], ssem, add=True)

        for (buf, sem, ssem, jj) in ((rows_a, sem_a, sem_sa, j),
                                     (rows_b, sem_b, sem_sb, j + 1)):
          pltpu.make_async_copy(buf, acc.at[dst_v.at[jj]], ssem).wait()

          @pl.when(jj + 2 < CH)
          def _():
            pltpu.async_copy(h_hbm.at[src_v.at[jj + 2]], buf, sem)

    plsc.subcore_barrier()
    pltpu.sync_copy(acc.at[pl.ds(sid * RPT, RPT)],
                    out_hbm.at[cid, pl.ds(sid * RPT, RPT)])

  kern = pl.kernel(
      body,
      out_type=jax.ShapeDtypeStruct((NC, np_rows, D), jnp.float32),
      mesh=mesh,
      scratch_types=[
          pltpu.VMEM((CH, B_E), jnp.int32),
          pltpu.VMEM((CH, B_E), jnp.int32),
          pltpu.VMEM((CH, B_E), jnp.float32),
          pltpu.VMEM((B_E, D), jnp.float32),
          pltpu.VMEM((B_E, D), jnp.float32),
          pltpu.VMEM_SHARED((np_rows, D), jnp.float32),
          pltpu.SemaphoreType.DMA,
          pltpu.SemaphoreType.DMA,
          pltpu.SemaphoreType.DMA,
          pltpu.SemaphoreType.DMA,
      ],
  )
  return kern(h, src3, dst3, w3)


def _tc_layer(parts, W, b2d, block_n):
  _, N, Din = parts.shape
  Hout = W.shape[1]
  assert N % block_n == 0

  def body(p_ref, w_ref, b_ref, o_ref):
    x = p_ref[0] + p_ref[1]
    y = jnp.dot(x, w_ref[...], preferred_element_type=jnp.float32) + b_ref[...]
    o_ref[...] = jnp.maximum(y, 0.0)

  return pl.pallas_call(
      body,
      grid=(N // block_n,),
      in_specs=[
          pl.BlockSpec((NC, block_n, Din), lambda i: (0, i, 0)),
          pl.BlockSpec((Din, Hout), lambda i: (0, 0)),
          pl.BlockSpec((1, Hout), lambda i: (0, 0)),
      ],
      out_specs=pl.BlockSpec((block_n, Hout), lambda i: (i, 0)),
      out_shape=jax.ShapeDtypeStruct((N, Hout), jnp.float32),
  )(parts, W, b2d)


def _tc_loss(parts, W2, b2d, labels2d, n_valid):
  _, NP, _ = parts.shape
  C = W2.shape[1]

  def body(p_ref, w_ref, b_ref, l_ref, o_ref):
    x = jnp.dot(p_ref[0] + p_ref[1], w_ref[...],
                preferred_element_type=jnp.float32) + b_ref[...]
    m = jnp.max(x, axis=1, keepdims=True)
    lse = jnp.log(jnp.sum(jnp.exp(x - m), axis=1, keepdims=True)) + m
    ids = lax.broadcasted_iota(jnp.int32, (NP, C), 1)
    picked = jnp.sum(jnp.where(ids == l_ref[...], x, 0.0), axis=1,
                     keepdims=True)
    rows = lax.broadcasted_iota(jnp.int32, (NP, 1), 0)
    nll = jnp.where(rows < n_valid, lse - picked, 0.0)
    o_ref[...] = jnp.sum(nll, keepdims=True) / n_valid

  out = pl.pallas_call(
      body,
      out_shape=jax.ShapeDtypeStruct((1, 1), jnp.float32),
  )(parts, W2, b2d, labels2d)
  return out[0, 0]


@jax.jit
def kernel(features, edge_index, edge_weight, labels, W0, b0, W1, b1, W2, b2):
  N = features.shape[0]
  E = edge_weight.shape[0]
  NP = NS * 128 * -(--(-N // NS) // 128)

  NB = -(-(-(-E // NW)) // B_E)
  NB += NB % 2
  total = NW * NB * B_E
  pad = total - E
  src3 = jnp.pad(edge_index[0], (0, pad)).reshape(NW, NB, B_E)
  dst3 = jnp.pad(edge_index[1], (0, pad)).reshape(NW, NB, B_E)
  w3 = jnp.pad(edge_weight, (0, pad)).reshape(NW, NB, B_E)

  labels_p = jnp.pad(labels.astype(jnp.int32), (0, NP - N)).reshape(-1, 1)

  a0 = _seg_sum(features, src3, dst3, w3, NP)
  h1 = _tc_layer(a0, W0, b0.reshape(1, -1), 2048)
  a1 = _seg_sum(h1, src3, dst3, w3, NP)
  h2 = _tc_layer(a1, W1, b1.reshape(1, -1), 2048)
  a2 = _seg_sum(h2, src3, dst3, w3, NP)
  return _tc_loss(a2, W2, b2.reshape(1, -1), labels_p, N)
